# baseline (speedup 1.0000x reference)
import functools

import jax
import jax.numpy as jnp
from jax import lax
from jax.experimental import pallas as pl
from jax.experimental.pallas import tpu as pltpu
from jax.experimental.pallas import tpu_sc as plsc

NC = 2
NS = 16
LANES = 16
ZCH = 25


def _make_sc_agg(n_nodes, d_row, e_total, ch):
    nw = NC * NS
    ept = e_total // nw
    nch = ept // ch
    nch2 = nch // 2
    rpt = n_nodes // NS
    rq = rpt // ZCH

    mesh = plsc.VectorSubcoreMesh(core_axis_name="c", subcore_axis_name="s")

    @functools.partial(
        pl.kernel,
        out_type=jax.ShapeDtypeStruct((NC * n_nodes, d_row), jnp.float32),
        mesh=mesh,
        scratch_types=[
            pltpu.VMEM((nch2, ch), jnp.int32),
            pltpu.VMEM((nch2, ch), jnp.int32),
            pltpu.VMEM((ch, d_row), jnp.float32),
            pltpu.VMEM((ch, d_row), jnp.float32),
            pltpu.VMEM_SHARED((n_nodes, d_row), jnp.float32),
            pltpu.SemaphoreType.DMA,
            pltpu.SemaphoreType.DMA,
            pltpu.SemaphoreType.DMA,
            pltpu.SemaphoreType.DMA,
        ],
        compiler_params=pltpu.CompilerParams(use_tc_tiling_on_sc=False),
    )
    def sc_agg(feat_hbm, src_hbm, dst_hbm, out_hbm,
               idx_s, idx_d, rows0, rows1, agg, gs0, gs1, ss0, ss1):
        c = lax.axis_index("c")
        s = lax.axis_index("s")
        w = c * NS + s

        def zero_row(r, carry):
            for k in range(d_row // LANES):
                rows0[r, pl.ds(k * LANES, LANES)] = jnp.zeros(
                    (LANES,), jnp.float32)
            return carry
        lax.fori_loop(0, ZCH, zero_row, 0)
        zrow = rows0.at[pl.ds(0, ZCH)]
        for q in range(rq):
            pltpu.sync_copy(zrow, agg.at[pl.ds(s * rpt + q * ZCH, ZCH)])
        plsc.subcore_barrier()

        for half in range(2):
            pltpu.sync_copy(
                src_hbm.at[pl.ds(w * nch + half * nch2, nch2)], idx_s)
            pltpu.sync_copy(
                dst_hbm.at[pl.ds(w * nch + half * nch2, nch2)], idx_d)

            pltpu.async_copy(feat_hbm.at[idx_s.at[0]], rows0, gs0)
            pltpu.async_copy(feat_hbm.at[idx_s.at[1]], rows1, gs1)

            def step(t, carry):
                j0 = 2 * t
                pltpu.make_async_copy(
                    feat_hbm.at[idx_s.at[j0]], rows0, gs0).wait()
                pltpu.async_copy(rows0, agg.at[idx_d.at[j0]], ss0, add=True)
                pltpu.make_async_copy(
                    feat_hbm.at[idx_s.at[j0 + 1]], rows1, gs1).wait()
                pltpu.async_copy(
                    rows1, agg.at[idx_d.at[j0 + 1]], ss1, add=True)

                @pl.when(t < nch2 // 2 - 1)
                def _():
                    pltpu.make_async_copy(
                        rows0, agg.at[idx_d.at[j0]], ss0).wait()
                    pltpu.async_copy(
                        feat_hbm.at[idx_s.at[j0 + 2]], rows0, gs0)
                    pltpu.make_async_copy(
                        rows1, agg.at[idx_d.at[j0 + 1]], ss1).wait()
                    pltpu.async_copy(
                        feat_hbm.at[idx_s.at[j0 + 3]], rows1, gs1)
                return carry

            lax.fori_loop(0, nch2 // 2, step, 0)
            pltpu.make_async_copy(
                rows0, agg.at[idx_d.at[nch2 - 2]], ss0).wait()
            pltpu.make_async_copy(
                rows1, agg.at[idx_d.at[nch2 - 1]], ss1).wait()

        plsc.subcore_barrier()

        pltpu.sync_copy(
            agg.at[pl.ds(s * rpt, rpt)],
            out_hbm.at[pl.ds(c * n_nodes + s * rpt, rpt)])

    return sc_agg


def _combine1(p1, x, Wl, bl, Wr):
    n, d = x.shape
    dr = p1.shape[1]
    bm = 400
    grid = n // bm

    def body(pa, pb, xr, wl, b, wr, h_ref, inv_ref):
        agg = pa[:, :d] + pb[:, :d]
        cnt = pa[:, d:d + 1] + pb[:, d:d + 1]
        inv = 1.0 / jnp.maximum(cnt, 1.0)
        mean = agg * inv
        mm = lax.dot_general(mean, wl[...], (((1,), (1,)), ((), ())),
                             preferred_element_type=jnp.float32)
        mm2 = lax.dot_general(xr[...], wr[...], (((1,), (1,)), ((), ())),
                              preferred_element_type=jnp.float32)
        res = mm + b[...] + mm2
        h_ref[...] = jnp.maximum(res, 0.0)
        inv_ref[...] = jnp.broadcast_to(inv, (bm, 8))

    return pl.pallas_call(
        body,
        grid=(grid,),
        in_specs=[
            pl.BlockSpec((bm, dr), lambda i: (i, 0)),
            pl.BlockSpec((bm, dr), lambda i, g=grid: (i + g, 0)),
            pl.BlockSpec((bm, d), lambda i: (i, 0)),
            pl.BlockSpec((d, d), lambda i: (0, 0)),
            pl.BlockSpec((1, d), lambda i: (0, 0)),
            pl.BlockSpec((d, d), lambda i: (0, 0)),
        ],
        out_specs=[
            pl.BlockSpec((bm, d), lambda i: (i, 0)),
            pl.BlockSpec((bm, 8), lambda i: (i, 0)),
        ],
        out_shape=[
            jax.ShapeDtypeStruct((n, d), jnp.float32),
            jax.ShapeDtypeStruct((n, 8), jnp.float32),
        ],
    )(p1, p1, x, Wl, bl, Wr)


def _combine2(p2, h, inv8, Wl, bl, Wr):
    n, d = h.shape
    bm = 400
    grid = n // bm

    def body(pa, pb, hr, invr, wl, b, wr, out_ref):
        mean = (pa[...] + pb[...]) * invr[:, 0:1]
        mm = lax.dot_general(mean, wl[...], (((1,), (1,)), ((), ())),
                             preferred_element_type=jnp.float32)
        mm2 = lax.dot_general(hr[...], wr[...], (((1,), (1,)), ((), ())),
                              preferred_element_type=jnp.float32)
        out_ref[...] = mm + b[...] + mm2

    return pl.pallas_call(
        body,
        grid=(grid,),
        in_specs=[
            pl.BlockSpec((bm, d), lambda i: (i, 0)),
            pl.BlockSpec((bm, d), lambda i, g=grid: (i + g, 0)),
            pl.BlockSpec((bm, d), lambda i: (i, 0)),
            pl.BlockSpec((bm, 8), lambda i: (i, 0)),
            pl.BlockSpec((d, d), lambda i: (0, 0)),
            pl.BlockSpec((1, d), lambda i: (0, 0)),
            pl.BlockSpec((d, d), lambda i: (0, 0)),
        ],
        out_specs=pl.BlockSpec((bm, d), lambda i: (i, 0)),
        out_shape=jax.ShapeDtypeStruct((n, d), jnp.float32),
    )(p2, p2, h, inv8, Wl, bl, Wr)


def kernel(x, edge_index, W1l, b1l, W1r, W2l, b2l, W2r):
    n, d = x.shape
    e = edge_index.shape[1]
    d_aug = d + LANES
    ch1, ch2 = 100, 125

    src1 = edge_index[0].reshape(e // ch1, ch1)
    dst1 = edge_index[1].reshape(e // ch1, ch1)
    src2 = edge_index[0].reshape(e // ch2, ch2)
    dst2 = edge_index[1].reshape(e // ch2, ch2)

    x_aug = jnp.concatenate(
        [x, jnp.ones((n, 1), jnp.float32),
         jnp.zeros((n, LANES - 1), jnp.float32)], axis=1)

    p1 = _make_sc_agg(n, d_aug, e, ch1)(x_aug, src1, dst1)
    h, inv8 = _combine1(p1, x, W1l, b1l.reshape(1, d), W1r)
    p2 = _make_sc_agg(n, d, e, ch2)(h, src2, dst2)
    out = _combine2(p2, h, inv8, W2l, b2l.reshape(1, d), W2r)
    return out


# speedup vs baseline: 1.2092x; 1.2092x over previous
import functools

import jax
import jax.numpy as jnp
from jax import lax
from jax.experimental import pallas as pl
from jax.experimental.pallas import tpu as pltpu
from jax.experimental.pallas import tpu_sc as plsc

NC = 2
NS = 16
LANES = 16
ZCH = 25


def _make_sc_agg(n_nodes, d_row, e_total, ch):
    nw = NC * NS
    ept = e_total // nw
    nch = ept // ch
    nch2 = nch // 2
    rpt = n_nodes // NS
    rq = rpt // ZCH

    mesh = plsc.VectorSubcoreMesh(core_axis_name="c", subcore_axis_name="s")

    @functools.partial(
        pl.kernel,
        out_type=jax.ShapeDtypeStruct((NC * n_nodes, d_row), jnp.float32),
        mesh=mesh,
        scratch_types=[
            pltpu.VMEM((nch2, ch), jnp.int32),
            pltpu.VMEM((nch2, ch), jnp.int32),
            pltpu.VMEM((ch, d_row), jnp.float32),
            pltpu.VMEM((ch, d_row), jnp.float32),
            pltpu.VMEM_SHARED((n_nodes, d_row), jnp.float32),
            pltpu.SemaphoreType.DMA,
            pltpu.SemaphoreType.DMA,
        ],
        compiler_params=pltpu.CompilerParams(use_tc_tiling_on_sc=False),
    )
    def sc_agg(feat_hbm, src_hbm, dst_hbm, out_hbm,
               idx_s, idx_d, rows0, rows1, agg, gs0, gs1):
        c = lax.axis_index("c")
        s = lax.axis_index("s")
        w = c * NS + s

        def zero_row(r, carry):
            for k in range(d_row // LANES):
                rows0[r, pl.ds(k * LANES, LANES)] = jnp.zeros(
                    (LANES,), jnp.float32)
            return carry
        lax.fori_loop(0, ZCH, zero_row, 0)
        zrow = rows0.at[pl.ds(0, ZCH)]
        for q in range(rq):
            pltpu.sync_copy(zrow, agg.at[pl.ds(s * rpt + q * ZCH, ZCH)])
        plsc.subcore_barrier()

        for half in range(2):
            pltpu.sync_copy(
                src_hbm.at[pl.ds(w * nch + half * nch2, nch2)], idx_s)
            pltpu.sync_copy(
                dst_hbm.at[pl.ds(w * nch + half * nch2, nch2)], idx_d)

            pltpu.async_copy(feat_hbm.at[idx_s.at[0]], rows0, gs0)

            def step(t, carry):
                j0 = 2 * t
                pltpu.async_copy(feat_hbm.at[idx_s.at[j0 + 1]], rows1, gs1)
                pltpu.make_async_copy(
                    feat_hbm.at[idx_s.at[j0]], rows0, gs0).wait()
                pltpu.sync_copy(rows0, agg.at[idx_d.at[j0]], add=True)

                @pl.when(t < nch2 // 2 - 1)
                def _():
                    pltpu.async_copy(
                        feat_hbm.at[idx_s.at[j0 + 2]], rows0, gs0)

                pltpu.make_async_copy(
                    feat_hbm.at[idx_s.at[j0 + 1]], rows1, gs1).wait()
                pltpu.sync_copy(rows1, agg.at[idx_d.at[j0 + 1]], add=True)
                return carry

            lax.fori_loop(0, nch2 // 2, step, 0)

        plsc.subcore_barrier()

        pltpu.sync_copy(
            agg.at[pl.ds(s * rpt, rpt)],
            out_hbm.at[pl.ds(c * n_nodes + s * rpt, rpt)])

    return sc_agg


def _combine1(p1, x, Wl, bl, Wr):
    n, d = x.shape
    dr = p1.shape[1]
    bm = 400
    grid = n // bm

    def body(pa, pb, xr, wl, b, wr, h_ref, inv_ref):
        agg = pa[:, :d] + pb[:, :d]
        cnt = pa[:, d:d + 1] + pb[:, d:d + 1]
        inv = 1.0 / jnp.maximum(cnt, 1.0)
        mean = agg * inv
        mm = lax.dot_general(mean, wl[...], (((1,), (1,)), ((), ())),
                             preferred_element_type=jnp.float32)
        mm2 = lax.dot_general(xr[...], wr[...], (((1,), (1,)), ((), ())),
                              preferred_element_type=jnp.float32)
        res = mm + b[...] + mm2
        h_ref[...] = jnp.maximum(res, 0.0)
        inv_ref[...] = jnp.broadcast_to(inv, (bm, 8))

    return pl.pallas_call(
        body,
        grid=(grid,),
        in_specs=[
            pl.BlockSpec((bm, dr), lambda i: (i, 0)),
            pl.BlockSpec((bm, dr), lambda i, g=grid: (i + g, 0)),
            pl.BlockSpec((bm, d), lambda i: (i, 0)),
            pl.BlockSpec((d, d), lambda i: (0, 0)),
            pl.BlockSpec((1, d), lambda i: (0, 0)),
            pl.BlockSpec((d, d), lambda i: (0, 0)),
        ],
        out_specs=[
            pl.BlockSpec((bm, d), lambda i: (i, 0)),
            pl.BlockSpec((bm, 8), lambda i: (i, 0)),
        ],
        out_shape=[
            jax.ShapeDtypeStruct((n, d), jnp.float32),
            jax.ShapeDtypeStruct((n, 8), jnp.float32),
        ],
    )(p1, p1, x, Wl, bl, Wr)


def _combine2(p2, h, inv8, Wl, bl, Wr):
    n, d = h.shape
    bm = 400
    grid = n // bm

    def body(pa, pb, hr, invr, wl, b, wr, out_ref):
        mean = (pa[...] + pb[...]) * invr[:, 0:1]
        mm = lax.dot_general(mean, wl[...], (((1,), (1,)), ((), ())),
                             preferred_element_type=jnp.float32)
        mm2 = lax.dot_general(hr[...], wr[...], (((1,), (1,)), ((), ())),
                              preferred_element_type=jnp.float32)
        out_ref[...] = mm + b[...] + mm2

    return pl.pallas_call(
        body,
        grid=(grid,),
        in_specs=[
            pl.BlockSpec((bm, d), lambda i: (i, 0)),
            pl.BlockSpec((bm, d), lambda i, g=grid: (i + g, 0)),
            pl.BlockSpec((bm, d), lambda i: (i, 0)),
            pl.BlockSpec((bm, 8), lambda i: (i, 0)),
            pl.BlockSpec((d, d), lambda i: (0, 0)),
            pl.BlockSpec((1, d), lambda i: (0, 0)),
            pl.BlockSpec((d, d), lambda i: (0, 0)),
        ],
        out_specs=pl.BlockSpec((bm, d), lambda i: (i, 0)),
        out_shape=jax.ShapeDtypeStruct((n, d), jnp.float32),
    )(p2, p2, h, inv8, Wl, bl, Wr)


def kernel(x, edge_index, W1l, b1l, W1r, W2l, b2l, W2r):
    n, d = x.shape
    e = edge_index.shape[1]
    d_aug = d + LANES
    ch1, ch2 = 100, 125

    src1 = edge_index[0].reshape(e // ch1, ch1)
    dst1 = edge_index[1].reshape(e // ch1, ch1)
    src2 = edge_index[0].reshape(e // ch2, ch2)
    dst2 = edge_index[1].reshape(e // ch2, ch2)

    x_aug = jnp.concatenate(
        [x, jnp.ones((n, 1), jnp.float32),
         jnp.zeros((n, LANES - 1), jnp.float32)], axis=1)

    p1 = _make_sc_agg(n, d_aug, e, ch1)(x_aug, src1, dst1)
    h, inv8 = _combine1(p1, x, W1l, b1l.reshape(1, d), W1r)
    p2 = _make_sc_agg(n, d, e, ch2)(h, src2, dst2)
    out = _combine2(p2, h, inv8, W2l, b2l.reshape(1, d), W2r)
    return out
